# baseline (speedup 1.0000x reference)
import jax
import jax.numpy as jnp
from jax import lax
from jax.experimental import pallas as pl
from jax.experimental.pallas import tpu as pltpu
from jax.experimental.pallas import tpu_sc as plsc

_B = 16384
_D = 64
_NC = 2
_NS = 16
_NW = _NC * _NS
_BPW = _B // _NW
_CH = 128
_NCH = _BPW // _CH
_L = 16
_W = 2 * _D


def _sqrt_vec(x):
    i = lax.bitcast_convert_type(x, jnp.int32)
    i = jnp.int32(0x5F3759DF) - lax.shift_right_arithmetic(i, 1)
    y = lax.bitcast_convert_type(i, jnp.float32)
    half_x = x * 0.5
    for _ in range(3):
        y = y * (1.5 - half_x * y * y)
    return jnp.where(x > 0.0, x * y, 0.0)


def _body(heads_hbm, rels_hbm, tails_hbm, ent_hbm, rel_hbm, out_hbm,
          hidx, ridx, tidx, hoff, roff, toff, hbuf, rbuf, tbuf, outv, sem):
    cid = lax.axis_index("c")
    sid = lax.axis_index("s")
    wid = sid * _NC + cid
    base = wid * _BPW

    lanes = lax.iota(jnp.int32, _L)
    perms = [(lanes + s) & (_L - 1) for s in (8, 4, 2, 1)]
    dnums = lax.GatherDimensionNumbers(
        offset_dims=(), collapsed_slice_dims=(0,), start_index_map=(0,))

    def perm(v, p):
        return lax.gather(v, p[:, None], dnums, (1,),
                          mode=lax.GatherScatterMode.PROMISE_IN_BOUNDS)

    def hsum(v):
        for p in perms:
            v = v + perm(v, p)
        return v

    def chunk(c, carry):
        off = base + c * _CH
        pltpu.sync_copy(heads_hbm.at[pl.ds(off, _CH)], hidx)
        pltpu.sync_copy(rels_hbm.at[pl.ds(off, _CH)], ridx)
        pltpu.sync_copy(tails_hbm.at[pl.ds(off, _CH)], tidx)
        for g in range(_CH // _L):
            sl = pl.ds(g * _L, _L)
            for ibuf, obuf in ((hidx, hoff), (ridx, roff), (tidx, toff)):
                v = ibuf[sl]
                obuf[sl] = lax.shift_left(v & 1, 6)
                ibuf[sl] = lax.shift_right_logical(v, 1)
        cps = [pltpu.async_copy(ent_hbm.at[hidx], hbuf, sem),
               pltpu.async_copy(rel_hbm.at[ridx], rbuf, sem),
               pltpu.async_copy(ent_hbm.at[tidx], tbuf, sem)]
        for cp in cps:
            cp.wait()
        for g in range(_CH // _L):
            sl = pl.ds(g * _L, _L)
            ohv, orv, otv = hoff[sl], roff[sl], toff[sl]
            totals = jnp.zeros((_L,), jnp.float32)
            for l in range(_L):
                e = g * _L + l
                oh = lax.squeeze(lax.slice(ohv, (l,), (l + 1,)), (0,))
                o_r = lax.squeeze(lax.slice(orv, (l,), (l + 1,)), (0,))
                ot = lax.squeeze(lax.slice(otv, (l,), (l + 1,)), (0,))
                acc = jnp.zeros((_L,), jnp.float32)
                for k in range(_D // _L):
                    h = hbuf[e, pl.ds(oh + k * _L, _L)]
                    r = rbuf[e, pl.ds(o_r + k * _L, _L)]
                    t = tbuf[e, pl.ds(ot + k * _L, _L)]
                    s = (h + r) - t
                    acc = acc + s * s
                totals = jnp.where(lanes == l, hsum(acc), totals)
            outv[pl.ds(c * _CH + g * _L, _L)] = _sqrt_vec(totals)
        return carry

    lax.fori_loop(0, _NCH, chunk, 0)
    pltpu.sync_copy(outv, out_hbm.at[pl.ds(base, _BPW)])


def kernel(heads, relations, tails, entity_emb, relation_emb):
    ent2 = jnp.reshape(entity_emb, (entity_emb.shape[0] // 2, _W))
    rel2 = jnp.reshape(relation_emb, (relation_emb.shape[0] // 2, _W))
    mesh = plsc.VectorSubcoreMesh(core_axis_name="c", subcore_axis_name="s")
    f = pl.kernel(
        _body,
        mesh=mesh,
        compiler_params=pltpu.CompilerParams(use_tc_tiling_on_sc=True),
        out_type=jax.ShapeDtypeStruct((_B,), jnp.float32),
        scratch_types=[
            pltpu.VMEM((_CH,), jnp.int32),
            pltpu.VMEM((_CH,), jnp.int32),
            pltpu.VMEM((_CH,), jnp.int32),
            pltpu.VMEM((_CH,), jnp.int32),
            pltpu.VMEM((_CH,), jnp.int32),
            pltpu.VMEM((_CH,), jnp.int32),
            pltpu.VMEM((_CH, _W), jnp.float32),
            pltpu.VMEM((_CH, _W), jnp.float32),
            pltpu.VMEM((_CH, _W), jnp.float32),
            pltpu.VMEM((_BPW,), jnp.float32),
            pltpu.SemaphoreType.DMA,
        ],
    )
    return f(heads, relations, tails, ent2, rel2)


# speedup vs baseline: 1.7231x; 1.7231x over previous
import jax
import jax.numpy as jnp
from jax import lax
from jax.experimental import pallas as pl
from jax.experimental.pallas import tpu as pltpu
from jax.experimental.pallas import tpu_sc as plsc

_B = 16384
_D = 64
_NC = 2
_NS = 16
_NW = _NC * _NS
_BPW = _B // _NW
_CH = 128
_NCH = _BPW // _CH
_L = 16


def _sqrt_vec(x):
    i = lax.bitcast_convert_type(x, jnp.int32)
    i = jnp.int32(0x5F3759DF) - lax.shift_right_arithmetic(i, 1)
    y = lax.bitcast_convert_type(i, jnp.float32)
    half_x = x * 0.5
    for _ in range(3):
        y = y * (1.5 - half_x * y * y)
    return jnp.where(x > 0.0, x * y, 0.0)


def _body(heads_hbm, rels_hbm, tails_hbm, ent_hbm, rel_hbm, out_hbm,
          hidx, ridx, tidx, hbuf, rbuf, tbuf, outv, sem):
    cid = lax.axis_index("c")
    sid = lax.axis_index("s")
    wid = sid * _NC + cid
    base = wid * _BPW

    pltpu.sync_copy(heads_hbm.at[pl.ds(base, _BPW)], hidx)
    pltpu.sync_copy(rels_hbm.at[pl.ds(base, _BPW)], ridx)
    pltpu.sync_copy(tails_hbm.at[pl.ds(base, _BPW)], tidx)

    lanes = lax.iota(jnp.int32, _L)
    perms = [(lanes + s) & (_L - 1) for s in (8, 4, 2, 1)]
    dnums = lax.GatherDimensionNumbers(
        offset_dims=(), collapsed_slice_dims=(0,), start_index_map=(0,))

    def perm(v, p):
        return lax.gather(v, p[:, None], dnums, (1,),
                          mode=lax.GatherScatterMode.PROMISE_IN_BOUNDS)

    def hsum(v):
        for p in perms:
            v = v + perm(v, p)
        return v

    def chunk(c, carry):
        cbase = c * _CH

        def issue(g, carry2):
            sl = pl.ds(cbase + g * _L, _L)
            hv, rv, tv = hidx[sl], ridx[sl], tidx[sl]
            for l in range(_L):
                e = g * _L + l
                pltpu.async_copy(ent_hbm.at[hv[l]], hbuf.at[e], sem)
                pltpu.async_copy(rel_hbm.at[rv[l]], rbuf.at[e], sem)
                pltpu.async_copy(ent_hbm.at[tv[l]], tbuf.at[e], sem)
            return carry2

        lax.fori_loop(0, _CH // _L, issue, 0)

        pltpu.make_async_copy(ent_hbm.at[pl.ds(0, _CH)], hbuf, sem).wait()
        pltpu.make_async_copy(rel_hbm.at[pl.ds(0, _CH)], rbuf, sem).wait()
        pltpu.make_async_copy(ent_hbm.at[pl.ds(0, _CH)], tbuf, sem).wait()

        def group(g, carry2):
            totals = jnp.zeros((_L,), jnp.float32)
            for l in range(_L):
                e = g * _L + l
                acc = jnp.zeros((_L,), jnp.float32)
                for k in range(_D // _L):
                    cs = pl.ds(k * _L, _L)
                    h = hbuf[e, cs]
                    r = rbuf[e, cs]
                    t = tbuf[e, cs]
                    s = (h + r) - t
                    acc = acc + s * s
                totals = jnp.where(lanes == l, hsum(acc), totals)
            outv[pl.ds(cbase + g * _L, _L)] = _sqrt_vec(totals)
            return carry2

        lax.fori_loop(0, _CH // _L, group, 0)
        return carry

    lax.fori_loop(0, _NCH, chunk, 0)
    pltpu.sync_copy(outv, out_hbm.at[pl.ds(base, _BPW)])


def kernel(heads, relations, tails, entity_emb, relation_emb):
    mesh = plsc.VectorSubcoreMesh(core_axis_name="c", subcore_axis_name="s")
    f = pl.kernel(
        _body,
        mesh=mesh,
        compiler_params=pltpu.CompilerParams(use_tc_tiling_on_sc=True),
        out_type=jax.ShapeDtypeStruct((_B,), jnp.float32),
        scratch_types=[
            pltpu.VMEM((_BPW,), jnp.int32),
            pltpu.VMEM((_BPW,), jnp.int32),
            pltpu.VMEM((_BPW,), jnp.int32),
            pltpu.VMEM((_CH, _D), jnp.float32),
            pltpu.VMEM((_CH, _D), jnp.float32),
            pltpu.VMEM((_CH, _D), jnp.float32),
            pltpu.VMEM((_BPW,), jnp.float32),
            pltpu.SemaphoreType.DMA,
        ],
    )
    return f(heads, relations, tails, entity_emb, relation_emb)
